# baseline (speedup 1.0000x reference)
import jax
import jax.numpy as jnp
from jax.experimental import pallas as pl
from jax.experimental.pallas import tpu as pltpu

_LEAD = 1024
_N_CHUNKS = 16


def _copy_body(x_ref, o_ref, sems):
    for i in range(_N_CHUNKS):
        pltpu.make_async_copy(x_ref.at[i], o_ref.at[i], sems.at[i]).start()
    for i in range(_N_CHUNKS):
        pltpu.make_async_copy(x_ref.at[i], o_ref.at[i], sems.at[i]).wait()


def kernel(inputs):
    x = inputs.reshape(_N_CHUNKS, _LEAD // _N_CHUNKS, 200, 176)
    out = pl.pallas_call(
        _copy_body,
        out_shape=jax.ShapeDtypeStruct(x.shape, x.dtype),
        in_specs=[pl.BlockSpec(memory_space=pltpu.MemorySpace.HBM)],
        out_specs=pl.BlockSpec(memory_space=pltpu.MemorySpace.HBM),
        scratch_shapes=[pltpu.SemaphoreType.DMA((_N_CHUNKS,))],
    )(x)
    return out.reshape(inputs.shape)


# speedup vs baseline: 15.9769x; 15.9769x over previous
import jax
import jax.numpy as jnp
from jax.experimental import pallas as pl
from jax.experimental.pallas import tpu as pltpu

_N_CHUNKS = 64
_ROWS = 16
_SLOTS = 8
_LOOKAHEAD = 4


def _copy_body(x_ref, o_ref, buf, in_sems, out_sems):
    N, K, D = _N_CHUNKS, _SLOTS, _LOOKAHEAD

    def in_copy(c, slot):
        return pltpu.make_async_copy(x_ref.at[c], buf.at[slot], in_sems.at[slot])

    def out_copy(c, slot):
        return pltpu.make_async_copy(buf.at[slot], o_ref.at[c], out_sems.at[slot])

    for j in range(D):
        in_copy(j, j).start()

    def body(i, carry):
        slot = jax.lax.rem(i, K)
        in_copy(i, slot).wait()
        out_copy(i, slot).start()
        nxt = i + D

        @pl.when(nxt < N)
        def _():
            nslot = jax.lax.rem(nxt, K)

            @pl.when(nxt >= K)
            def _():
                out_copy(nxt - K, nslot).wait()

            in_copy(nxt, nslot).start()

        return carry

    jax.lax.fori_loop(0, N, body, 0)
    for c in range(N - K, N):
        out_copy(c, c % K).wait()


def kernel(inputs):
    x = inputs.reshape(_N_CHUNKS, _ROWS, 200, 176)
    out = pl.pallas_call(
        _copy_body,
        out_shape=jax.ShapeDtypeStruct(x.shape, x.dtype),
        in_specs=[pl.BlockSpec(memory_space=pltpu.MemorySpace.HBM)],
        out_specs=pl.BlockSpec(memory_space=pltpu.MemorySpace.HBM),
        scratch_shapes=[
            pltpu.VMEM((_SLOTS, _ROWS, 200, 176), jnp.float32),
            pltpu.SemaphoreType.DMA((_SLOTS,)),
            pltpu.SemaphoreType.DMA((_SLOTS,)),
        ],
    )(x)
    return out.reshape(inputs.shape)
